# Initial kernel scaffold; baseline (speedup 1.0000x reference)
import jax
import jax.numpy as jnp
from jax.experimental import pallas as pl


def kernel(x, edge_index, W1, b1, g1, be1, W2, b2, g2, be2, eps):
    raise NotImplementedError("write your pallas kernel here")


# speedup vs baseline: 5.1059x; 5.1059x over previous
import jax
import jax.numpy as jnp
from jax import lax
from jax.experimental import pallas as pl
from jax.experimental.pallas import tpu as pltpu
from jax.experimental.pallas import tpu_sc as plsc

N = 10000
E = 160000
D = 256
H = D // 2
NT = 16
K = 128
NCH = E // K
NITER = (NCH + NT - 1) // NT
NP = 10240
RP = NP // NT
RPC = 128
BN_EPS = 1e-5


def _agg_body(xh, ei, out, ebuf, ibuf, gbuf, zbuf, acc, sem):
    cid = lax.axis_index("c")
    sid = lax.axis_index("s")

    zero16 = jnp.zeros((16,), jnp.float32)

    @pl.loop(0, RPC)
    def _zero(r):
        for j in range(H // 16):
            zbuf[r, pl.ds(j * 16, 16)] = zero16

    for i in range(RP // RPC):
        pltpu.sync_copy(zbuf, acc.at[pl.ds(sid * RP + i * RPC, RPC)])
    plsc.subcore_barrier()

    @pl.loop(0, NITER)
    def _chunk(i):
        ch = sid + i * NT

        @pl.when(ch < NCH)
        def _():
            pltpu.sync_copy(ei.at[:, pl.ds(ch * K, K)], ebuf)
            for j in range(K // 16):
                sl = pl.ds(j * 16, 16)
                ibuf[sl] = ebuf[0, sl] * 2 + cid
            pltpu.async_copy(xh.at[ibuf], gbuf, sem).wait()
            pltpu.sync_copy(gbuf, acc.at[ebuf.at[1]], add=True)

    plsc.subcore_barrier()

    for i in range(RP // RPC):
        row = sid * RP + i * RPC
        pltpu.sync_copy(acc.at[pl.ds(row, RPC)], zbuf)
        pltpu.sync_copy(zbuf, out.at[pl.ds(cid * NP + row, RPC)])


_agg_call = pl.kernel(
    _agg_body,
    out_type=jax.ShapeDtypeStruct((2 * NP, H), jnp.float32),
    mesh=plsc.VectorSubcoreMesh(core_axis_name="c", subcore_axis_name="s"),
    scratch_types=[
        pltpu.VMEM((2, K), jnp.int32),
        pltpu.VMEM((K,), jnp.int32),
        pltpu.VMEM((K, H), jnp.float32),
        pltpu.VMEM((RPC, H), jnp.float32),
        pltpu.VMEM_SHARED((NP, H), jnp.float32),
        pltpu.SemaphoreType.DMA,
    ],
)


def _mlp_body(x_ref, agg_ref, w1_ref, b1_ref, g1_ref, be1_ref,
              w2_ref, b2_ref, g2_ref, be2_ref, eps_ref, out_ref):
    x = x_ref[...]
    agg = jnp.concatenate([agg_ref[:N, :], agg_ref[NP:NP + N, :]], axis=1)
    h = (1.0 + eps_ref[0, 0]) * x + agg

    y = jnp.dot(h, w1_ref[...], preferred_element_type=jnp.float32) + b1_ref[...]
    mean = jnp.mean(y, axis=0, keepdims=True)
    var = jnp.mean((y - mean) ** 2, axis=0, keepdims=True)
    y = (y - mean) * lax.rsqrt(var + BN_EPS) * g1_ref[...] + be1_ref[...]
    y = jnp.maximum(y, 0.0)

    z = jnp.dot(y, w2_ref[...], preferred_element_type=jnp.float32) + b2_ref[...]
    mean2 = jnp.mean(z, axis=0, keepdims=True)
    var2 = jnp.mean((z - mean2) ** 2, axis=0, keepdims=True)
    z = (z - mean2) * lax.rsqrt(var2 + BN_EPS) * g2_ref[...] + be2_ref[...]

    out_ref[...] = z + x


_mlp_call = pl.pallas_call(
    _mlp_body,
    out_shape=jax.ShapeDtypeStruct((N, D), jnp.float32),
    in_specs=[pl.BlockSpec(memory_space=pltpu.VMEM)] * 10
    + [pl.BlockSpec(memory_space=pltpu.SMEM)],
)


def kernel(x, edge_index, W1, b1, g1, be1, W2, b2, g2, be2, eps):
    xh = x.reshape(2 * N, H)
    agg2 = _agg_call(xh, edge_index)

    return _mlp_call(
        x, agg2, W1,
        b1.reshape(1, D), g1.reshape(1, D), be1.reshape(1, D),
        W2, b2.reshape(1, D), g2.reshape(1, D), be2.reshape(1, D),
        eps.reshape(1, 1),
    )


# speedup vs baseline: 7.8674x; 1.5408x over previous
import jax
import jax.numpy as jnp
from jax import lax
from jax.experimental import pallas as pl
from jax.experimental.pallas import tpu as pltpu
from jax.experimental.pallas import tpu_sc as plsc

N = 10000
E = 160000
D = 256
H = D // 2
NT = 16
K = 128
NCH = E // K
NITER = (NCH + NT - 1) // NT
NP = 10112
RP = NP // NT
RPC = 128
_STAGE = [(0, 128), (128, 128), (256, 128), (384, 128), (512, 120)]
BN_EPS = 1e-5


def _agg_body(xh, ei, out, e0, i0, g0, e1, i1, g1, zbuf, acc, sem0, sem1):
    cid = lax.axis_index("c")
    sid = lax.axis_index("s")

    zero16 = jnp.zeros((16,), jnp.float32)

    @pl.loop(0, RPC)
    def _zero(r):
        for j in range(H // 16):
            zbuf[r, pl.ds(j * 16, 16)] = zero16

    for off, n in _STAGE:
        pltpu.sync_copy(zbuf.at[pl.ds(0, n)], acc.at[pl.ds(sid * RP + off, n)])
    plsc.subcore_barrier()

    bufs = ((e0, i0, g0, sem0), (e1, i1, g1, sem1))

    def pred(i):
        return (i < NITER) & ((sid + i * NT) < NCH)

    def issue(i, b):
        eb, ib, gb, sem = bufs[b]

        @pl.when(pred(i))
        def _():
            ch = sid + i * NT
            pltpu.sync_copy(ei.at[:, pl.ds(ch * K, K)], eb)
            for j in range(K // 16):
                sl = pl.ds(j * 16, 16)
                ib[sl] = eb[0, sl] * 2 + cid
            pltpu.async_copy(xh.at[ib], gb, sem)

    def drain(i, b):
        eb, ib, gb, sem = bufs[b]

        @pl.when(pred(i))
        def _():
            pltpu.make_async_copy(xh.at[ib], gb, sem).wait()
            pltpu.sync_copy(gb, acc.at[eb.at[1]], add=True)

    issue(0, 0)

    @pl.loop(0, NITER // 2 + 1)
    def _main(it):
        i = it * 2
        issue(i + 1, 1)
        drain(i, 0)
        issue(i + 2, 0)
        drain(i + 1, 1)

    plsc.subcore_barrier()

    for off, n in _STAGE:
        row = sid * RP + off
        pltpu.sync_copy(acc.at[pl.ds(row, n)], zbuf.at[pl.ds(0, n)])
        pltpu.sync_copy(zbuf.at[pl.ds(0, n)], out.at[pl.ds(cid * NP + row, n)])


_agg_call = pl.kernel(
    _agg_body,
    out_type=jax.ShapeDtypeStruct((2 * NP, H), jnp.float32),
    mesh=plsc.VectorSubcoreMesh(core_axis_name="c", subcore_axis_name="s"),
    scratch_types=[
        pltpu.VMEM((2, K), jnp.int32),
        pltpu.VMEM((K,), jnp.int32),
        pltpu.VMEM((K, H), jnp.float32),
        pltpu.VMEM((2, K), jnp.int32),
        pltpu.VMEM((K,), jnp.int32),
        pltpu.VMEM((K, H), jnp.float32),
        pltpu.VMEM((RPC, H), jnp.float32),
        pltpu.VMEM_SHARED((NP, H), jnp.float32),
        pltpu.SemaphoreType.DMA,
        pltpu.SemaphoreType.DMA,
    ],
)


def _mlp_body(x_ref, agg_ref, w1_ref, b1_ref, g1_ref, be1_ref,
              w2_ref, b2_ref, g2_ref, be2_ref, eps_ref, out_ref):
    x = x_ref[...]
    agg = jnp.concatenate([agg_ref[:N, :], agg_ref[NP:NP + N, :]], axis=1)
    h = (1.0 + eps_ref[0, 0]) * x + agg

    y = jnp.dot(h, w1_ref[...], preferred_element_type=jnp.float32) + b1_ref[...]
    mean = jnp.mean(y, axis=0, keepdims=True)
    var = jnp.mean((y - mean) ** 2, axis=0, keepdims=True)
    y = (y - mean) * lax.rsqrt(var + BN_EPS) * g1_ref[...] + be1_ref[...]
    y = jnp.maximum(y, 0.0)

    z = jnp.dot(y, w2_ref[...], preferred_element_type=jnp.float32) + b2_ref[...]
    mean2 = jnp.mean(z, axis=0, keepdims=True)
    var2 = jnp.mean((z - mean2) ** 2, axis=0, keepdims=True)
    z = (z - mean2) * lax.rsqrt(var2 + BN_EPS) * g2_ref[...] + be2_ref[...]

    out_ref[...] = z + x


_mlp_call = pl.pallas_call(
    _mlp_body,
    out_shape=jax.ShapeDtypeStruct((N, D), jnp.float32),
    in_specs=[pl.BlockSpec(memory_space=pltpu.VMEM)] * 10
    + [pl.BlockSpec(memory_space=pltpu.SMEM)],
)


def kernel(x, edge_index, W1, b1, g1, be1, W2, b2, g2, be2, eps):
    xh = x.reshape(2 * N, H)
    agg2 = _agg_call(xh, edge_index)

    return _mlp_call(
        x, agg2, W1,
        b1.reshape(1, D), g1.reshape(1, D), be1.reshape(1, D),
        W2, b2.reshape(1, D), g2.reshape(1, D), be2.reshape(1, D),
        eps.reshape(1, 1),
    )


# speedup vs baseline: 8.7252x; 1.1090x over previous
import jax
import jax.numpy as jnp
from jax import lax
from jax.experimental import pallas as pl
from jax.experimental.pallas import tpu as pltpu
from jax.experimental.pallas import tpu_sc as plsc

N = 10000
E = 160000
D = 256
H = D // 2
NT = 16
K = 128
NCH = E // K
NITER = (NCH + NT - 1) // NT
NP = 10112
RP = NP // NT
RPC = 128
_STAGE = [(0, 128), (128, 128), (256, 128), (384, 128), (512, 120)]
BN_EPS = 1e-5


def _agg_body(xh, ei, out, e0, g0, e1, g1, e2, g2, acc,
              se0, sg0, se1, sg1, se2, sg2):
    cid = lax.axis_index("c")
    sid = lax.axis_index("s")

    zero16 = jnp.zeros((16,), jnp.float32)

    @pl.loop(0, RPC)
    def _zero(r):
        for j in range(H // 16):
            g0[r, pl.ds(j * 16, 16)] = zero16

    for off, n in _STAGE:
        pltpu.sync_copy(g0.at[pl.ds(0, n)], acc.at[pl.ds(sid * RP + off, n)])
    plsc.subcore_barrier()

    bufs = ((e0, g0, se0, sg0), (e1, g1, se1, sg1), (e2, g2, se2, sg2))

    def pred(i):
        return (i < NITER) & ((sid + i * NT) < NCH)

    def eslice(i):
        return ei.at[:, pl.ds((sid + i * NT) * K, K)]

    def start_idx(i, b):
        eb, gb, sem_e, sem_g = bufs[b]

        @pl.when(pred(i))
        def _():
            pltpu.async_copy(eslice(i), eb, sem_e)

    def start_gather(i, b):
        eb, gb, sem_e, sem_g = bufs[b]

        @pl.when(pred(i))
        def _():
            pltpu.make_async_copy(eslice(i), eb, sem_e).wait()
            for j in range(K // 16):
                sl = pl.ds(j * 16, 16)
                eb[0, sl] = eb[0, sl] * 2 + cid
            pltpu.async_copy(xh.at[eb.at[0]], gb, sem_g)

    def drain(i, b):
        eb, gb, sem_e, sem_g = bufs[b]

        @pl.when(pred(i))
        def _():
            pltpu.make_async_copy(xh.at[eb.at[0]], gb, sem_g).wait()
            pltpu.sync_copy(gb, acc.at[eb.at[1]], add=True)

    start_idx(0, 0)
    start_idx(1, 1)
    start_gather(0, 0)

    @pl.loop(0, NITER // 3 + 1)
    def _main(it):
        i = it * 3
        for b in range(3):
            start_idx(i + b + 2, (b + 2) % 3)
            start_gather(i + b + 1, (b + 1) % 3)
            drain(i + b, b)

    plsc.subcore_barrier()

    for off, n in _STAGE:
        row = sid * RP + off
        pltpu.sync_copy(acc.at[pl.ds(row, n)], g0.at[pl.ds(0, n)])
        pltpu.sync_copy(g0.at[pl.ds(0, n)], out.at[pl.ds(cid * NP + row, n)])


_agg_call = pl.kernel(
    _agg_body,
    out_type=jax.ShapeDtypeStruct((2 * NP, H), jnp.float32),
    mesh=plsc.VectorSubcoreMesh(core_axis_name="c", subcore_axis_name="s"),
    scratch_types=[
        pltpu.VMEM((2, K), jnp.int32),
        pltpu.VMEM((K, H), jnp.float32),
        pltpu.VMEM((2, K), jnp.int32),
        pltpu.VMEM((K, H), jnp.float32),
        pltpu.VMEM((2, K), jnp.int32),
        pltpu.VMEM((K, H), jnp.float32),
        pltpu.VMEM_SHARED((NP, H), jnp.float32),
        pltpu.SemaphoreType.DMA,
        pltpu.SemaphoreType.DMA,
        pltpu.SemaphoreType.DMA,
        pltpu.SemaphoreType.DMA,
        pltpu.SemaphoreType.DMA,
        pltpu.SemaphoreType.DMA,
    ],
)


def _mlp_body(x_ref, agg_ref, w1_ref, b1_ref, g1_ref, be1_ref,
              w2_ref, b2_ref, g2_ref, be2_ref, eps_ref, out_ref):
    x = x_ref[...]
    agg = jnp.concatenate([agg_ref[:N, :], agg_ref[NP:NP + N, :]], axis=1)
    h = (1.0 + eps_ref[0, 0]) * x + agg

    y = jnp.dot(h, w1_ref[...], preferred_element_type=jnp.float32) + b1_ref[...]
    mean = jnp.mean(y, axis=0, keepdims=True)
    var = jnp.mean((y - mean) ** 2, axis=0, keepdims=True)
    y = (y - mean) * lax.rsqrt(var + BN_EPS) * g1_ref[...] + be1_ref[...]
    y = jnp.maximum(y, 0.0)

    z = jnp.dot(y, w2_ref[...], preferred_element_type=jnp.float32) + b2_ref[...]
    mean2 = jnp.mean(z, axis=0, keepdims=True)
    var2 = jnp.mean((z - mean2) ** 2, axis=0, keepdims=True)
    z = (z - mean2) * lax.rsqrt(var2 + BN_EPS) * g2_ref[...] + be2_ref[...]

    out_ref[...] = z + x


_mlp_call = pl.pallas_call(
    _mlp_body,
    out_shape=jax.ShapeDtypeStruct((N, D), jnp.float32),
    in_specs=[pl.BlockSpec(memory_space=pltpu.VMEM)] * 10
    + [pl.BlockSpec(memory_space=pltpu.SMEM)],
)


def kernel(x, edge_index, W1, b1, g1, be1, W2, b2, g2, be2, eps):
    xh = x.reshape(2 * N, H)
    agg2 = _agg_call(xh, edge_index)

    return _mlp_call(
        x, agg2, W1,
        b1.reshape(1, D), g1.reshape(1, D), be1.reshape(1, D),
        W2, b2.reshape(1, D), g2.reshape(1, D), be2.reshape(1, D),
        eps.reshape(1, 1),
    )


# speedup vs baseline: 8.7661x; 1.0047x over previous
import jax
import jax.numpy as jnp
from jax import lax
from jax.experimental import pallas as pl
from jax.experimental.pallas import tpu as pltpu
from jax.experimental.pallas import tpu_sc as plsc

N = 10000
E = 160000
D = 256
H = D // 2
NT = 16
K = 128
NCH = E // K
NITER = (NCH + NT - 1) // NT
NP = 10112
RP = NP // NT
RPC = 128
_STAGE = [(0, 128), (128, 128), (256, 128), (384, 128), (512, 120)]
BN_EPS = 1e-5


def _agg_body(xh, ei, out, e0, g0, e1, g1, e2, g2, acc,
              se0, sg0, se1, sg1, se2, sg2):
    cid = lax.axis_index("c")
    sid = lax.axis_index("s")

    zero16 = jnp.zeros((16,), jnp.float32)

    @pl.loop(0, RPC)
    def _zero(r):
        for j in range(H // 16):
            g0[r, pl.ds(j * 16, 16)] = zero16

    for off, n in _STAGE:
        pltpu.sync_copy(g0.at[pl.ds(0, n)], acc.at[pl.ds(sid * RP + off, n)])
    plsc.subcore_barrier()

    bufs = ((e0, g0, se0, sg0), (e1, g1, se1, sg1), (e2, g2, se2, sg2))

    def pred(i):
        return (i < NITER) & ((sid + i * NT) < NCH)

    def eslice(i):
        return ei.at[:, pl.ds((sid + i * NT) * K, K)]

    def start_idx(i, b):
        eb, gb, sem_e, sem_g = bufs[b]

        @pl.when(pred(i))
        def _():
            pltpu.async_copy(eslice(i), eb, sem_e)

    def start_gather(i, b):
        eb, gb, sem_e, sem_g = bufs[b]

        @pl.when(pred(i))
        def _():
            pltpu.make_async_copy(eslice(i), eb, sem_e).wait()
            for j in range(K // 16):
                sl = pl.ds(j * 16, 16)
                eb[0, sl] = eb[0, sl] * 2 + cid
            pltpu.async_copy(xh.at[eb.at[0]], gb, sem_g)

    def drain(i, b):
        eb, gb, sem_e, sem_g = bufs[b]

        @pl.when(pred(i))
        def _():
            pltpu.make_async_copy(xh.at[eb.at[0]], gb, sem_g).wait()
            pltpu.sync_copy(gb, acc.at[eb.at[1]], add=True)

    start_idx(0, 0)
    start_idx(1, 1)
    start_gather(0, 0)

    @pl.loop(0, NITER // 3 + 1)
    def _main(it):
        i = it * 3
        for b in range(3):
            start_idx(i + b + 2, (b + 2) % 3)
            start_gather(i + b + 1, (b + 1) % 3)
            drain(i + b, b)

    plsc.subcore_barrier()

    for off, n in _STAGE:
        row = sid * RP + off
        pltpu.sync_copy(acc.at[pl.ds(row, n)], g0.at[pl.ds(0, n)])
        pltpu.sync_copy(g0.at[pl.ds(0, n)], out.at[pl.ds(cid * NP + row, n)])


_agg_call = pl.kernel(
    _agg_body,
    out_type=jax.ShapeDtypeStruct((2 * NP, H), jnp.float32),
    mesh=plsc.VectorSubcoreMesh(core_axis_name="c", subcore_axis_name="s"),
    scratch_types=[
        pltpu.VMEM((2, K), jnp.int32),
        pltpu.VMEM((K, H), jnp.float32),
        pltpu.VMEM((2, K), jnp.int32),
        pltpu.VMEM((K, H), jnp.float32),
        pltpu.VMEM((2, K), jnp.int32),
        pltpu.VMEM((K, H), jnp.float32),
        pltpu.VMEM_SHARED((NP, H), jnp.float32),
        pltpu.SemaphoreType.DMA,
        pltpu.SemaphoreType.DMA,
        pltpu.SemaphoreType.DMA,
        pltpu.SemaphoreType.DMA,
        pltpu.SemaphoreType.DMA,
        pltpu.SemaphoreType.DMA,
    ],
)


def _mlp_body(x_ref, agg_ref, w1_ref, b1_ref, g1_ref, be1_ref,
              w2_ref, b2_ref, g2_ref, be2_ref, eps_ref, out_ref):
    x = x_ref[...]
    agg = jnp.concatenate([agg_ref[:N, :], agg_ref[NP:NP + N, :]], axis=1)
    h = (1.0 + eps_ref[0, 0]) * x + agg

    y = jnp.dot(h.astype(jnp.bfloat16), w1_ref[...].astype(jnp.bfloat16),
                preferred_element_type=jnp.float32) + b1_ref[...]
    mean = jnp.mean(y, axis=0, keepdims=True)
    var = jnp.mean((y - mean) ** 2, axis=0, keepdims=True)
    y = (y - mean) * lax.rsqrt(var + BN_EPS) * g1_ref[...] + be1_ref[...]
    y = jnp.maximum(y, 0.0)

    z = jnp.dot(y.astype(jnp.bfloat16), w2_ref[...].astype(jnp.bfloat16),
                preferred_element_type=jnp.float32) + b2_ref[...]
    mean2 = jnp.mean(z, axis=0, keepdims=True)
    var2 = jnp.mean((z - mean2) ** 2, axis=0, keepdims=True)
    z = (z - mean2) * lax.rsqrt(var2 + BN_EPS) * g2_ref[...] + be2_ref[...]

    out_ref[...] = z + x


_mlp_call = pl.pallas_call(
    _mlp_body,
    out_shape=jax.ShapeDtypeStruct((N, D), jnp.float32),
    in_specs=[pl.BlockSpec(memory_space=pltpu.VMEM)] * 10
    + [pl.BlockSpec(memory_space=pltpu.SMEM)],
)


def kernel(x, edge_index, W1, b1, g1, be1, W2, b2, g2, be2, eps):
    xh = x.reshape(2 * N, H)
    agg2 = _agg_call(xh, edge_index)

    return _mlp_call(
        x, agg2, W1,
        b1.reshape(1, D), g1.reshape(1, D), be1.reshape(1, D),
        W2, b2.reshape(1, D), g2.reshape(1, D), be2.reshape(1, D),
        eps.reshape(1, 1),
    )
